# baseline (speedup 1.0000x reference)
import functools

import jax
import jax.numpy as jnp
from jax import lax
from jax.experimental import pallas as pl
from jax.experimental.pallas import tpu as pltpu
from jax.experimental.pallas import tpu_sc as plsc

F = 26
V = 100000
D = 32
B = 16384

NC = 2
NS = 16
NW = NC * NS
LANES = 16
H0 = 50048
H1 = V - H0
HALF = (H0, H1)
LO = (0, H0)
PIECES0 = (12544, 12544, 12544, H0 - 3 * 12544)
POFF0 = (0, 12544, 25088, 37632)
IDS_CHUNK = 4096
NIDC = B // IDS_CHUNK


def _body(ids_hbm, tab_hbm, out_hbm, col0, col1, ids2, acc_v,
          cs0, cs1, is0, is1):
  c = lax.axis_index("s") * NC + lax.axis_index("c")
  col = (col0, col1)
  csem = (cs0, cs1)
  isem = (is0, is1)

  def col_piece0(f, q):
    off = POFF0[q]
    n = PIECES0[q]
    pltpu.async_copy(tab_hbm.at[f, c, pl.ds(off, n)],
                     col0.at[pl.ds(off, n)], cs0)

  def col_start1(f):
    pltpu.async_copy(tab_hbm.at[f, c, pl.ds(H0, H1)], col1, cs1)

  def col_wait(f, h):
    if h == 0:
      for q in range(4):
        off = POFF0[q]
        n = PIECES0[q]
        pltpu.make_async_copy(tab_hbm.at[f, c, pl.ds(off, n)],
                              col0.at[pl.ds(off, n)], cs0).wait()
    else:
      pltpu.make_async_copy(tab_hbm.at[f, c, pl.ds(H0, H1)], col1,
                            cs1).wait()

  def ids_start(f, j):
    p = j % 2
    pltpu.async_copy(ids_hbm.at[f, pl.ds(j * IDS_CHUNK, IDS_CHUNK)],
                     ids2.at[p], isem[p])

  def ids_wait(f, j):
    p = j % 2
    pltpu.make_async_copy(ids_hbm.at[f, pl.ds(j * IDS_CHUNK, IDS_CHUNK)],
                          ids2.at[p], isem[p]).wait()

  zeros = jnp.zeros((LANES,), jnp.float32)

  @plsc.parallel_loop(0, B, step=LANES, unroll=8)
  def _zinit(off):
    acc_v[pl.ds(off, LANES)] = zeros

  f0 = jnp.int32(0)
  for q in range(4):
    col_piece0(f0, q)
  ids_start(f0, 0)
  ids_start(f0, 1)

  SCHED = (
      ((0, True, 2, False), (1, True, 3, False),
       (2, True, None, False), (3, True, None, False)),
      ((2, False, 0, False), (3, False, 1, False),
       (0, True, 0, True), (1, True, 1, True)),
  )

  def field_body(f, carry):
    for h in (0, 1):
      col_wait(f, h)

      def next_piece(q, h=h, f=f):
        if h == 1:
          @pl.when(f + 1 < F)
          def _():
            col_piece0(f + 1, q)

      if h == 0:
        col_start1(f)
      else:
        next_piece(0)

      for pos, (j, wait, refill, nxt) in enumerate(SCHED[h]):
        p = j % 2
        if wait:
          ids_wait(f, j)
        jbase = j * IDS_CHUNK

        @plsc.parallel_loop(0, IDS_CHUNK, step=LANES, unroll=8)
        def _blk(off, h=h, p=p, jbase=jbase):
          vidx = ids2[p, pl.ds(off, LANES)]
          vloc = vidx - LO[h] if h else vidx
          m = vloc.astype(jnp.uint32) < jnp.uint32(HALF[h])
          g = plsc.load_gather(col[h], [vloc], mask=m)
          plsc.addupdate(acc_v.at[pl.ds(jbase + off, LANES)],
                         jnp.where(m, g, 0.0))

        if refill is not None:
          if nxt:
            @pl.when(f + 1 < F)
            def _():
              ids_start(f + 1, refill)
          else:
            ids_start(f, refill)
        if pos < 3:
          next_piece(pos + 1)

    return carry

  lax.fori_loop(0, F, field_body, 0)
  pltpu.sync_copy(acc_v, out_hbm.at[c])


@jax.jit
def kernel(ids, tables):
  ids_t = ids.T
  tab_t = tables.transpose(0, 2, 1)

  mesh = plsc.VectorSubcoreMesh(core_axis_name="c", subcore_axis_name="s")
  run = pl.kernel(
      _body,
      out_type=jax.ShapeDtypeStruct((D, B), jnp.float32),
      mesh=mesh,
      scratch_types=[
          pltpu.VMEM((H0,), jnp.float32),
          pltpu.VMEM((H1,), jnp.float32),
          pltpu.VMEM((2, IDS_CHUNK), jnp.int32),
          pltpu.VMEM((B,), jnp.float32),
          pltpu.SemaphoreType.DMA,
          pltpu.SemaphoreType.DMA,
          pltpu.SemaphoreType.DMA,
          pltpu.SemaphoreType.DMA,
      ],
      compiler_params=pltpu.CompilerParams(needs_layout_passes=False),
  )
  return run(ids_t, tab_t).T


# speedup vs baseline: 1.0668x; 1.0668x over previous
import functools

import jax
import jax.numpy as jnp
from jax import lax
from jax.experimental import pallas as pl
from jax.experimental.pallas import tpu as pltpu
from jax.experimental.pallas import tpu_sc as plsc

F = 26
V = 100000
D = 32
B = 16384

NC = 2
NS = 16
NW = NC * NS
LANES = 16
H0 = 50048
H1 = V - H0
HALF = (H0, H1)
LO = (0, H0)
IDS_CHUNK = 4096
NIDC = B // IDS_CHUNK


def _body(ids_hbm, tab_hbm, out_hbm, col0, col1, ids2, acc_v,
          cs0, cs1, is0, is1):
  c = lax.axis_index("s") * NC + lax.axis_index("c")
  col = (col0, col1)
  csem = (cs0, cs1)
  isem = (is0, is1)

  def col_start(f, h):
    pltpu.async_copy(tab_hbm.at[f, c, pl.ds(LO[h], HALF[h])], col[h], csem[h])

  def col_wait(f, h):
    pltpu.make_async_copy(tab_hbm.at[f, c, pl.ds(LO[h], HALF[h])], col[h],
                          csem[h]).wait()

  def ids_start(f, j):
    p = j % 2
    pltpu.async_copy(ids_hbm.at[pl.ds(f * B + j * IDS_CHUNK, IDS_CHUNK)],
                     ids2.at[p], isem[p])

  def ids_wait(f, j):
    p = j % 2
    pltpu.make_async_copy(ids_hbm.at[pl.ds(f * B + j * IDS_CHUNK, IDS_CHUNK)],
                          ids2.at[p], isem[p]).wait()

  zeros = jnp.zeros((LANES,), jnp.float32)

  @plsc.parallel_loop(0, B, step=LANES, unroll=8)
  def _zinit(off):
    acc_v[pl.ds(off, LANES)] = zeros

  f0 = jnp.int32(0)
  col_start(f0, 0)
  ids_start(f0, 0)
  ids_start(f0, 1)

  SCHED = (
      ((0, True, 2, False), (1, True, 3, False),
       (2, True, None, False), (3, True, None, False)),
      ((2, False, 0, False), (3, False, 1, False),
       (0, True, 0, True), (1, True, 1, True)),
  )

  def field_body(f, carry):
    for h in (0, 1):
      if h == 0:
        col_start(f, 1)
      else:
        @pl.when(f + 1 < F)
        def _():
          col_start(f + 1, 0)
      col_wait(f, h)

      for j, wait, refill, nxt in SCHED[h]:
        p = j % 2
        if wait:
          ids_wait(f, j)
        jbase = j * IDS_CHUNK

        @plsc.parallel_loop(0, IDS_CHUNK, step=LANES, unroll=8)
        def _blk(off, h=h, p=p, jbase=jbase):
          vidx = ids2[p, pl.ds(off, LANES)]
          vloc = vidx - LO[h] if h else vidx
          m = vloc.astype(jnp.uint32) < jnp.uint32(HALF[h])
          g = plsc.load_gather(col[h], [vloc], mask=m)
          plsc.addupdate(acc_v.at[pl.ds(jbase + off, LANES)],
                         jnp.where(m, g, 0.0))

        if refill is not None:
          if nxt:
            @pl.when(f + 1 < F)
            def _():
              ids_start(f + 1, refill)
          else:
            ids_start(f, refill)

    return carry

  lax.fori_loop(0, F, field_body, 0)
  pltpu.sync_copy(acc_v, out_hbm.at[c])


@jax.jit
def kernel(ids, tables):
  ids_t = jnp.reshape(ids.T, (F * B,))
  tab_t = tables.transpose(0, 2, 1)

  mesh = plsc.VectorSubcoreMesh(core_axis_name="c", subcore_axis_name="s")
  run = pl.kernel(
      _body,
      out_type=jax.ShapeDtypeStruct((D, B), jnp.float32),
      mesh=mesh,
      scratch_types=[
          pltpu.VMEM((H0,), jnp.float32),
          pltpu.VMEM((H1,), jnp.float32),
          pltpu.VMEM((2, IDS_CHUNK), jnp.int32),
          pltpu.VMEM((B,), jnp.float32),
          pltpu.SemaphoreType.DMA,
          pltpu.SemaphoreType.DMA,
          pltpu.SemaphoreType.DMA,
          pltpu.SemaphoreType.DMA,
      ],
      compiler_params=pltpu.CompilerParams(needs_layout_passes=False),
  )
  return run(ids_t, tab_t).T
